# baseline (speedup 1.0000x reference)
import functools

import jax
import jax.numpy as jnp
from jax import lax
from jax.experimental import pallas as pl
from jax.experimental.pallas import tpu as pltpu
from jax.experimental.pallas import tpu_sc as plsc

_N = 10000
_E = 160000
_DF = 256
_DH = 512

_NC = 2
_NS = 16
_LN = 16
_CB = 128

_NP = 10240
_EP = 163840
_ND = 10496
_TRASH = _NP


def _mesh():
    return plsc.VectorSubcoreMesh(
        core_axis_name="c", subcore_axis_name="s",
        num_cores=_NC, num_subcores=_NS)


_DEG_EPT = _EP // (_NC * _NS)
_DEG_G = 128
_DEG_ZR = _ND // _NS


def _deg_body(dst_hbm, degp_hbm, dst_v, idx_v, ones_v, zbuf_v, deg_sh):
    c = lax.axis_index("c")
    s = lax.axis_index("s")
    wid = c * _NS + s
    zero = jnp.zeros((_LN,), jnp.float32)
    for j in range(_DEG_ZR // _LN):
        zbuf_v[pl.ds(j * _LN, _LN)] = zero
    one = jnp.ones((_LN,), jnp.float32)
    for j in range(_DEG_G // _LN):
        ones_v[pl.ds(j * _LN, _LN)] = one
    pltpu.sync_copy(zbuf_v, deg_sh.at[pl.ds(s * _DEG_ZR, _DEG_ZR)])
    plsc.subcore_barrier()

    for r in range(_DEG_EPT // 1024):
        pltpu.sync_copy(
            dst_hbm.at[pl.ds(wid * _DEG_EPT + r * 1024, 1024)], dst_v)

        def body(g, carry):
            for j in range(_DEG_G // _LN):
                idx_v[pl.ds(j * _LN, _LN)] = \
                    dst_v[pl.ds(g * _DEG_G + j * _LN, _LN)]
            pltpu.sync_copy(ones_v, deg_sh.at[idx_v], add=True)
            return carry

        lax.fori_loop(0, 1024 // _DEG_G, body, 0)
    plsc.subcore_barrier()
    pltpu.sync_copy(deg_sh.at[pl.ds(s * _DEG_ZR, _DEG_ZR)], zbuf_v)
    pltpu.sync_copy(zbuf_v, degp_hbm.at[pl.ds(c * _ND + s * _DEG_ZR, _DEG_ZR)])


def _make_deg():
    return functools.partial(
        pl.kernel,
        out_type=jax.ShapeDtypeStruct((_NC * _ND,), jnp.float32),
        mesh=_mesh(),
        compiler_params=pltpu.CompilerParams(needs_layout_passes=False),
        scratch_types=[
            pltpu.VMEM((1024,), jnp.int32),
            pltpu.VMEM((_DEG_G,), jnp.int32),
            pltpu.VMEM((_DEG_G,), jnp.float32),
            pltpu.VMEM((_DEG_ZR,), jnp.float32),
            pltpu.VMEM_SHARED((_ND,), jnp.float32),
        ],
    )(_deg_body)


_deg_call = _make_deg()


def _make_agg(NB, C, K, G=32, RND=2048):
    ET = _EP // _NS
    R = C // _NS
    RG = 16 if R % 16 == 0 else 8
    NRND = ET // RND

    def body(src_hbm, dst_hbm, *refs):
        hs = refs[:NB]
        outs = refs[NB:2 * NB]
        ebs, ebd, lsrc, ldst, sidx, didx = refs[2 * NB:2 * NB + 6]
        stages = refs[2 * NB + 6:2 * NB + 6 + NB]
        accs = refs[2 * NB + 6 + NB:2 * NB + 6 + 2 * NB]
        sem = refs[-1]

        c = lax.axis_index("c")
        s = lax.axis_index("s")
        base = c * (K * C)
        r0 = s * R
        iota = jnp.arange(_LN, dtype=jnp.int32)

        for k in range(K):
            lo = base + k * C
            for q in range(R // RG):
                for b in range(NB):
                    pltpu.sync_copy(hs[b].at[pl.ds(lo + r0 + q * RG, RG)],
                                    stages[b].at[pl.ds(0, RG)])
                    pltpu.sync_copy(stages[b].at[pl.ds(0, RG)],
                                    accs[b].at[pl.ds(r0 + q * RG, RG)])
            plsc.subcore_barrier()

            for r in range(NRND):
                e0 = s * ET + r * RND
                pltpu.sync_copy(src_hbm.at[pl.ds(e0, RND)], ebs)
                pltpu.sync_copy(dst_hbm.at[pl.ds(e0, RND)], ebd)

                def cbody(i, cnt):
                    sv = ebs[pl.ds(i * _LN, _LN)]
                    dv = ebd[pl.ds(i * _LN, _LN)]
                    m = (dv >= lo) & (dv < lo + C)
                    inc = plsc.cumsum(jnp.where(m, 1, 0).astype(jnp.int32))
                    pos = cnt + inc - 1
                    plsc.store_scatter(lsrc, [pos], sv, mask=m)
                    plsc.store_scatter(ldst, [pos], dv - lo, mask=m)
                    return cnt + plsc.all_reduce_population_count(m)

                cnt = lax.fori_loop(0, RND // _LN, cbody,
                                    jnp.zeros((_LN,), jnp.int32))
                for j in range(G // _LN):
                    tail = cnt + iota + j * _LN
                    plsc.store_scatter(lsrc, [tail],
                                       jnp.zeros((_LN,), jnp.int32))
                    plsc.store_scatter(ldst, [tail],
                                       jnp.full((_LN,), C, jnp.int32))

                n = cnt[0]
                ng = (n + G - 1) // G

                def gbody(g, carry):
                    for j in range(G // _LN):
                        sidx[pl.ds(j * _LN, _LN)] = lsrc[pl.ds(g * G + j * _LN, _LN)]
                        didx[pl.ds(j * _LN, _LN)] = ldst[pl.ds(g * G + j * _LN, _LN)]
                    descs = [pltpu.async_copy(hs[b].at[sidx], stages[b], sem)
                             for b in range(NB)]
                    for d in descs:
                        d.wait()
                    for b in range(NB):
                        pltpu.sync_copy(stages[b], accs[b].at[didx], add=True)
                    return carry

                lax.fori_loop(0, ng, gbody, 0)

            plsc.subcore_barrier()
            for q in range(R // RG):
                for b in range(NB):
                    pltpu.sync_copy(accs[b].at[pl.ds(r0 + q * RG, RG)],
                                    stages[b].at[pl.ds(0, RG)])
                    pltpu.sync_copy(stages[b].at[pl.ds(0, RG)],
                                    outs[b].at[pl.ds(lo + r0 + q * RG, RG)])
            plsc.subcore_barrier()

    return functools.partial(
        pl.kernel,
        out_type=tuple(jax.ShapeDtypeStruct((_NP, _CB), jnp.float32)
                       for _ in range(NB)),
        mesh=_mesh(),
        compiler_params=pltpu.CompilerParams(needs_layout_passes=False),
        scratch_types=(
            [pltpu.VMEM((RND,), jnp.int32),
             pltpu.VMEM((RND,), jnp.int32),
             pltpu.VMEM((RND + G,), jnp.int32),
             pltpu.VMEM((RND + G,), jnp.int32),
             pltpu.VMEM((G,), jnp.int32),
             pltpu.VMEM((G,), jnp.int32)]
            + [pltpu.VMEM((G, _CB), jnp.float32) for _ in range(NB)]
            + [pltpu.VMEM_SHARED((C + 8, _CB), jnp.float32) for _ in range(NB)]
            + [pltpu.SemaphoreType.DMA]
        ),
    )(body)


_agg1_call = _make_agg(4, 1024, 5, 64)
_agg2_call = _make_agg(2, 1024, 5, 64)


_BR = 256


def _b1_kernel(x_ref, w_ref, d0_ref, d1_ref, h0, h1, h2, h3, dis_ref):
    deg = d0_ref[...] + d1_ref[...] + 1.0
    dis = lax.rsqrt(deg)
    dis_ref[...] = dis
    h = jnp.dot(x_ref[...] * dis, w_ref[...],
                preferred_element_type=jnp.float32)
    h0[...] = h[:, 0 * _CB:1 * _CB]
    h1[...] = h[:, 1 * _CB:2 * _CB]
    h2[...] = h[:, 2 * _CB:3 * _CB]
    h3[...] = h[:, 3 * _CB:4 * _CB]


def _b1_call(xp, W1, d0, d1):
    blk = pl.BlockSpec((_BR, _CB), lambda i: (i, 0))
    return pl.pallas_call(
        _b1_kernel,
        grid=(_NP // _BR,),
        in_specs=[
            pl.BlockSpec((_BR, _DF), lambda i: (i, 0)),
            pl.BlockSpec((_DF, _DH), lambda i: (0, 0)),
            pl.BlockSpec((_BR, 1), lambda i: (i, 0)),
            pl.BlockSpec((_BR, 1), lambda i: (i, 0)),
        ],
        out_specs=[blk, blk, blk, blk,
                   pl.BlockSpec((_BR, 1), lambda i: (i, 0))],
        out_shape=[jax.ShapeDtypeStruct((_NP, _CB), jnp.float32)] * 4
        + [jax.ShapeDtypeStruct((_NP, 1), jnp.float32)],
    )(xp, W1, d0, d1)


def _b2_kernel(a0, a1, a2, a3, dis_ref, b_ref, w_ref, o0, o1):
    dis = dis_ref[...]
    zs = []
    for b, a in enumerate((a0, a1, a2, a3)):
        t = dis * a[...] + b_ref[:, b * _CB:(b + 1) * _CB]
        zs.append(jnp.maximum(t, 0.0) * dis)
    z = jnp.concatenate(zs, axis=1)
    o = jnp.dot(z, w_ref[...], preferred_element_type=jnp.float32)
    o0[...] = o[:, 0 * _CB:1 * _CB]
    o1[...] = o[:, 1 * _CB:2 * _CB]


def _b2_call(agg1, dis, b1r, W2):
    blk = pl.BlockSpec((_BR, _CB), lambda i: (i, 0))
    return pl.pallas_call(
        _b2_kernel,
        grid=(_NP // _BR,),
        in_specs=[blk] * 4 + [
            pl.BlockSpec((_BR, 1), lambda i: (i, 0)),
            pl.BlockSpec((1, _DH), lambda i: (0, 0)),
            pl.BlockSpec((_DH, _DF), lambda i: (0, 0)),
        ],
        out_specs=[blk, blk],
        out_shape=[jax.ShapeDtypeStruct((_NP, _CB), jnp.float32)] * 2,
    )(*agg1, dis, b1r, W2)


def _b3_kernel(a0, a1, dis_ref, b_ref, o_ref):
    dis = dis_ref[...]
    o_ref[...] = jnp.concatenate(
        [dis * a0[...], dis * a1[...]], axis=1) + b_ref[...]


def _b3_call(agg2, dis, b2r):
    blk = pl.BlockSpec((_BR, _CB), lambda i: (i, 0))
    return pl.pallas_call(
        _b3_kernel,
        grid=(_NP // _BR,),
        in_specs=[blk] * 2 + [
            pl.BlockSpec((_BR, 1), lambda i: (i, 0)),
            pl.BlockSpec((1, _DF), lambda i: (0, 0)),
        ],
        out_specs=pl.BlockSpec((_BR, _DF), lambda i: (i, 0)),
        out_shape=jax.ShapeDtypeStruct((_NP, _DF), jnp.float32),
    )(*agg2, dis, b2r)


@jax.jit
def kernel(x, edge_index, W1, b1, W2, b2):
    src = edge_index[0].astype(jnp.int32)
    dst = edge_index[1].astype(jnp.int32)
    srcp = jnp.concatenate([src, jnp.zeros((_EP - _E,), jnp.int32)])
    dstp = jnp.concatenate([dst, jnp.full((_EP - _E,), _TRASH, jnp.int32)])
    xp = jnp.concatenate([x, jnp.zeros((_NP - _N, _DF), x.dtype)])

    degp = _deg_call(dstp)
    d0 = degp[:_NP, None]
    d1 = degp[_ND:_ND + _NP, None]

    *h1s, dis = _b1_call(xp, W1, d0, d1)
    agg1 = _agg1_call(srcp, dstp, *h1s)
    h2s = _b2_call(agg1, dis, b1[None, :], W2)
    agg2 = _agg2_call(srcp, dstp, *h2s)
    out = _b3_call(agg2, dis, b2[None, :])
    return out[:_N]


# speedup vs baseline: 3.3164x; 3.3164x over previous
import functools

import jax
import jax.numpy as jnp
from jax import lax
from jax.experimental import pallas as pl
from jax.experimental.pallas import tpu as pltpu
from jax.experimental.pallas import tpu_sc as plsc

_N = 10000
_E = 160000
_DF = 256
_DH = 512

_NC = 2
_NS = 16
_LN = 16
_CB = 128

_NP = 10240
_EP = 163840
_ND = 10496
_TRASH = _NP


def _mesh():
    return plsc.VectorSubcoreMesh(
        core_axis_name="c", subcore_axis_name="s",
        num_cores=_NC, num_subcores=_NS)


_DEG_EPT = _EP // (_NC * _NS)
_DEG_G = 128
_DEG_ZR = _ND // _NS


def _deg_body(dst_hbm, degp_hbm, dst_v, idx_v, ones_v, zbuf_v, deg_sh):
    c = lax.axis_index("c")
    s = lax.axis_index("s")
    wid = c * _NS + s
    zero = jnp.zeros((_LN,), jnp.float32)
    for j in range(_DEG_ZR // _LN):
        zbuf_v[pl.ds(j * _LN, _LN)] = zero
    one = jnp.ones((_LN,), jnp.float32)
    for j in range(_DEG_G // _LN):
        ones_v[pl.ds(j * _LN, _LN)] = one
    pltpu.sync_copy(zbuf_v, deg_sh.at[pl.ds(s * _DEG_ZR, _DEG_ZR)])
    plsc.subcore_barrier()

    for r in range(_DEG_EPT // 1024):
        pltpu.sync_copy(
            dst_hbm.at[pl.ds(wid * _DEG_EPT + r * 1024, 1024)], dst_v)

        def body(g, carry):
            for j in range(_DEG_G // _LN):
                idx_v[pl.ds(j * _LN, _LN)] = \
                    dst_v[pl.ds(g * _DEG_G + j * _LN, _LN)]
            pltpu.sync_copy(ones_v, deg_sh.at[idx_v], add=True)
            return carry

        lax.fori_loop(0, 1024 // _DEG_G, body, 0)
    plsc.subcore_barrier()
    pltpu.sync_copy(deg_sh.at[pl.ds(s * _DEG_ZR, _DEG_ZR)], zbuf_v)
    pltpu.sync_copy(zbuf_v, degp_hbm.at[pl.ds(c * _ND + s * _DEG_ZR, _DEG_ZR)])


def _make_deg():
    return functools.partial(
        pl.kernel,
        out_type=jax.ShapeDtypeStruct((_NC * _ND,), jnp.float32),
        mesh=_mesh(),
        compiler_params=pltpu.CompilerParams(needs_layout_passes=False),
        scratch_types=[
            pltpu.VMEM((1024,), jnp.int32),
            pltpu.VMEM((_DEG_G,), jnp.int32),
            pltpu.VMEM((_DEG_G,), jnp.float32),
            pltpu.VMEM((_DEG_ZR,), jnp.float32),
            pltpu.VMEM_SHARED((_ND,), jnp.float32),
        ],
    )(_deg_body)


_deg_call = _make_deg()


def _make_agg(NB, C, K, G=32, RND=2048):
    ET = _EP // _NS
    R = C // _NS
    RG = 16 if R % 16 == 0 else 8
    NRND = ET // RND

    def body(src_hbm, dst_hbm, *refs):
        hs = refs[:NB]
        outs = refs[NB:2 * NB]
        ebs, ebd, lsrc, ldst, sidx, didx = refs[2 * NB:2 * NB + 6]
        stages = refs[2 * NB + 6:2 * NB + 6 + NB]
        accs = refs[2 * NB + 6 + NB:2 * NB + 6 + 2 * NB]
        sem = refs[-1]

        c = lax.axis_index("c")
        s = lax.axis_index("s")
        base = c * (K * C)
        r0 = s * R
        iota = jnp.arange(_LN, dtype=jnp.int32)

        for k in range(K):
            lo = base + k * C
            for b in range(NB):
                pltpu.sync_copy(hs[b].at[pl.ds(lo + r0, R)],
                                accs[b].at[pl.ds(r0, R)])
            plsc.subcore_barrier()

            for r in range(NRND):
                e0 = s * ET + r * RND
                pltpu.sync_copy(src_hbm.at[pl.ds(e0, RND)], ebs)
                pltpu.sync_copy(dst_hbm.at[pl.ds(e0, RND)], ebd)

                def cbody(i, cnt):
                    sv = ebs[pl.ds(i * _LN, _LN)]
                    dv = ebd[pl.ds(i * _LN, _LN)]
                    m = (dv >= lo) & (dv < lo + C)
                    inc = plsc.cumsum(jnp.where(m, 1, 0).astype(jnp.int32))
                    pos = cnt + inc - 1
                    plsc.store_scatter(lsrc, [pos], sv, mask=m)
                    plsc.store_scatter(ldst, [pos], dv - lo, mask=m)
                    return cnt + plsc.all_reduce_population_count(m)

                cnt = lax.fori_loop(0, RND // _LN, cbody,
                                    jnp.zeros((_LN,), jnp.int32))
                for j in range(G // _LN):
                    tail = cnt + iota + j * _LN
                    plsc.store_scatter(lsrc, [tail], iota + j * _LN)
                    plsc.store_scatter(ldst, [tail], iota + C)

                n = cnt[0]
                ng = (n + G - 1) // G

                def gbody(g, carry):
                    for j in range(G // _LN):
                        sidx[pl.ds(j * _LN, _LN)] = lsrc[pl.ds(g * G + j * _LN, _LN)]
                        didx[pl.ds(j * _LN, _LN)] = ldst[pl.ds(g * G + j * _LN, _LN)]
                    descs = [pltpu.async_copy(hs[b].at[sidx], stages[b], sem)
                             for b in range(NB)]
                    for d in descs:
                        d.wait()
                    for b in range(NB):
                        pltpu.sync_copy(stages[b], accs[b].at[didx], add=True)
                    return carry

                lax.fori_loop(0, ng, gbody, 0)

            plsc.subcore_barrier()
            for b in range(NB):
                pltpu.sync_copy(accs[b].at[pl.ds(r0, R)],
                                outs[b].at[pl.ds(lo + r0, R)])
            plsc.subcore_barrier()

    return functools.partial(
        pl.kernel,
        out_type=tuple(jax.ShapeDtypeStruct((_NP, _CB), jnp.float32)
                       for _ in range(NB)),
        mesh=_mesh(),
        compiler_params=pltpu.CompilerParams(needs_layout_passes=False),
        scratch_types=(
            [pltpu.VMEM((RND,), jnp.int32),
             pltpu.VMEM((RND,), jnp.int32),
             pltpu.VMEM((RND + G,), jnp.int32),
             pltpu.VMEM((RND + G,), jnp.int32),
             pltpu.VMEM((G,), jnp.int32),
             pltpu.VMEM((G,), jnp.int32)]
            + [pltpu.VMEM((G, _CB), jnp.float32) for _ in range(NB)]
            + [pltpu.VMEM_SHARED((C + 8, _CB), jnp.float32) for _ in range(NB)]
            + [pltpu.SemaphoreType.DMA]
        ),
    )(body)


_agg1_call = _make_agg(4, 1024, 5, 64)
_agg2_call = _make_agg(2, 1024, 5, 64)


_BR = 256


def _b1_kernel(x_ref, w_ref, d0_ref, d1_ref, h0, h1, h2, h3, dis_ref):
    deg = d0_ref[...] + d1_ref[...] + 1.0
    dis = lax.rsqrt(deg)
    dis_ref[...] = dis
    h = jnp.dot(x_ref[...] * dis, w_ref[...],
                preferred_element_type=jnp.float32)
    h0[...] = h[:, 0 * _CB:1 * _CB]
    h1[...] = h[:, 1 * _CB:2 * _CB]
    h2[...] = h[:, 2 * _CB:3 * _CB]
    h3[...] = h[:, 3 * _CB:4 * _CB]


def _b1_call(xp, W1, d0, d1):
    blk = pl.BlockSpec((_BR, _CB), lambda i: (i, 0))
    return pl.pallas_call(
        _b1_kernel,
        grid=(_NP // _BR,),
        in_specs=[
            pl.BlockSpec((_BR, _DF), lambda i: (i, 0)),
            pl.BlockSpec((_DF, _DH), lambda i: (0, 0)),
            pl.BlockSpec((_BR, 1), lambda i: (i, 0)),
            pl.BlockSpec((_BR, 1), lambda i: (i, 0)),
        ],
        out_specs=[blk, blk, blk, blk,
                   pl.BlockSpec((_BR, 1), lambda i: (i, 0))],
        out_shape=[jax.ShapeDtypeStruct((_NP, _CB), jnp.float32)] * 4
        + [jax.ShapeDtypeStruct((_NP, 1), jnp.float32)],
    )(xp, W1, d0, d1)


def _b2_kernel(a0, a1, a2, a3, dis_ref, b_ref, w_ref, o0, o1):
    dis = dis_ref[...]
    zs = []
    for b, a in enumerate((a0, a1, a2, a3)):
        t = dis * a[...] + b_ref[:, b * _CB:(b + 1) * _CB]
        zs.append(jnp.maximum(t, 0.0) * dis)
    z = jnp.concatenate(zs, axis=1)
    o = jnp.dot(z, w_ref[...], preferred_element_type=jnp.float32)
    o0[...] = o[:, 0 * _CB:1 * _CB]
    o1[...] = o[:, 1 * _CB:2 * _CB]


def _b2_call(agg1, dis, b1r, W2):
    blk = pl.BlockSpec((_BR, _CB), lambda i: (i, 0))
    return pl.pallas_call(
        _b2_kernel,
        grid=(_NP // _BR,),
        in_specs=[blk] * 4 + [
            pl.BlockSpec((_BR, 1), lambda i: (i, 0)),
            pl.BlockSpec((1, _DH), lambda i: (0, 0)),
            pl.BlockSpec((_DH, _DF), lambda i: (0, 0)),
        ],
        out_specs=[blk, blk],
        out_shape=[jax.ShapeDtypeStruct((_NP, _CB), jnp.float32)] * 2,
    )(*agg1, dis, b1r, W2)


def _b3_kernel(a0, a1, dis_ref, b_ref, o_ref):
    dis = dis_ref[...]
    o_ref[...] = jnp.concatenate(
        [dis * a0[...], dis * a1[...]], axis=1) + b_ref[...]


def _b3_call(agg2, dis, b2r):
    blk = pl.BlockSpec((_BR, _CB), lambda i: (i, 0))
    return pl.pallas_call(
        _b3_kernel,
        grid=(_NP // _BR,),
        in_specs=[blk] * 2 + [
            pl.BlockSpec((_BR, 1), lambda i: (i, 0)),
            pl.BlockSpec((1, _DF), lambda i: (0, 0)),
        ],
        out_specs=pl.BlockSpec((_BR, _DF), lambda i: (i, 0)),
        out_shape=jax.ShapeDtypeStruct((_NP, _DF), jnp.float32),
    )(*agg2, dis, b2r)


@jax.jit
def kernel(x, edge_index, W1, b1, W2, b2):
    src = edge_index[0].astype(jnp.int32)
    dst = edge_index[1].astype(jnp.int32)
    srcp = jnp.concatenate([src, jnp.zeros((_EP - _E,), jnp.int32)])
    dstp = jnp.concatenate([dst, jnp.full((_EP - _E,), _TRASH, jnp.int32)])
    xp = jnp.concatenate([x, jnp.zeros((_NP - _N, _DF), x.dtype)])

    degp = _deg_call(dstp)
    d0 = degp[:_NP, None]
    d1 = degp[_ND:_ND + _NP, None]

    *h1s, dis = _b1_call(xp, W1, d0, d1)
    agg1 = _agg1_call(srcp, dstp, *h1s)
    h2s = _b2_call(agg1, dis, b1[None, :], W2)
    agg2 = _agg2_call(srcp, dstp, *h2s)
    out = _b3_call(agg2, dis, b2[None, :])
    return out[:_N]


# speedup vs baseline: 3.3174x; 1.0003x over previous
import functools

import jax
import jax.numpy as jnp
from jax import lax
from jax.experimental import pallas as pl
from jax.experimental.pallas import tpu as pltpu
from jax.experimental.pallas import tpu_sc as plsc

_N = 10000
_E = 160000
_DF = 256
_DH = 512

_NC = 2
_NS = 16
_LN = 16
_CB = 128

_NP = 10240
_EP = 163840
_ND = 10496
_TRASH = _NP


def _mesh():
    return plsc.VectorSubcoreMesh(
        core_axis_name="c", subcore_axis_name="s",
        num_cores=_NC, num_subcores=_NS)


_DEG_EPT = _EP // (_NC * _NS)
_DEG_G = 128
_DEG_ZR = _ND // _NS


def _deg_body(dst_hbm, degp_hbm, dst_v, idx_v, ones_v, zbuf_v, deg_sh):
    c = lax.axis_index("c")
    s = lax.axis_index("s")
    wid = c * _NS + s
    zero = jnp.zeros((_LN,), jnp.float32)
    for j in range(_DEG_ZR // _LN):
        zbuf_v[pl.ds(j * _LN, _LN)] = zero
    one = jnp.ones((_LN,), jnp.float32)
    for j in range(_DEG_G // _LN):
        ones_v[pl.ds(j * _LN, _LN)] = one
    pltpu.sync_copy(zbuf_v, deg_sh.at[pl.ds(s * _DEG_ZR, _DEG_ZR)])
    plsc.subcore_barrier()

    for r in range(_DEG_EPT // 1024):
        pltpu.sync_copy(
            dst_hbm.at[pl.ds(wid * _DEG_EPT + r * 1024, 1024)], dst_v)

        def body(g, carry):
            for j in range(_DEG_G // _LN):
                idx_v[pl.ds(j * _LN, _LN)] = \
                    dst_v[pl.ds(g * _DEG_G + j * _LN, _LN)]
            pltpu.sync_copy(ones_v, deg_sh.at[idx_v], add=True)
            return carry

        lax.fori_loop(0, 1024 // _DEG_G, body, 0)
    plsc.subcore_barrier()
    pltpu.sync_copy(deg_sh.at[pl.ds(s * _DEG_ZR, _DEG_ZR)], zbuf_v)
    pltpu.sync_copy(zbuf_v, degp_hbm.at[pl.ds(c * _ND + s * _DEG_ZR, _DEG_ZR)])


def _make_deg():
    return functools.partial(
        pl.kernel,
        out_type=jax.ShapeDtypeStruct((_NC * _ND,), jnp.float32),
        mesh=_mesh(),
        compiler_params=pltpu.CompilerParams(needs_layout_passes=False),
        scratch_types=[
            pltpu.VMEM((1024,), jnp.int32),
            pltpu.VMEM((_DEG_G,), jnp.int32),
            pltpu.VMEM((_DEG_G,), jnp.float32),
            pltpu.VMEM((_DEG_ZR,), jnp.float32),
            pltpu.VMEM_SHARED((_ND,), jnp.float32),
        ],
    )(_deg_body)


_deg_call = _make_deg()


def _make_agg(NB, C, K, G=32, RND=2048):
    ET = _EP // _NS
    R = C // _NS
    RG = 16 if R % 16 == 0 else 8
    NRND = ET // RND

    def body(src_hbm, dst_hbm, *refs):
        hs = refs[:NB]
        outs = refs[NB:2 * NB]
        ebs, ebd, lsrc, ldst, sidx, didx = refs[2 * NB:2 * NB + 6]
        stages = refs[2 * NB + 6:2 * NB + 6 + NB]
        accs = refs[2 * NB + 6 + NB:2 * NB + 6 + 2 * NB]
        sem = refs[-1]

        c = lax.axis_index("c")
        s = lax.axis_index("s")
        base = c * (K * C)
        r0 = s * R
        iota = jnp.arange(_LN, dtype=jnp.int32)

        for k in range(K):
            lo = base + k * C
            for b in range(NB):
                pltpu.sync_copy(hs[b].at[pl.ds(lo + r0, R)],
                                accs[b].at[pl.ds(r0, R)])
            plsc.subcore_barrier()

            for r in range(NRND):
                e0 = s * ET + r * RND
                pltpu.sync_copy(src_hbm.at[pl.ds(e0, RND)], ebs)
                pltpu.sync_copy(dst_hbm.at[pl.ds(e0, RND)], ebd)

                def cbody(i, cnt):
                    sv = ebs[pl.ds(i * _LN, _LN)]
                    dv = ebd[pl.ds(i * _LN, _LN)]
                    m = (dv >= lo) & (dv < lo + C)
                    inc = plsc.cumsum(jnp.where(m, 1, 0).astype(jnp.int32))
                    pos = cnt + inc - 1
                    plsc.store_scatter(lsrc, [pos], sv, mask=m)
                    plsc.store_scatter(ldst, [pos], dv - lo, mask=m)
                    return cnt + plsc.all_reduce_population_count(m)

                cnt = lax.fori_loop(0, RND // _LN, cbody,
                                    jnp.zeros((_LN,), jnp.int32))
                for j in range(G // _LN):
                    tail = cnt + iota + j * _LN
                    plsc.store_scatter(lsrc, [tail], iota + j * _LN)
                    plsc.store_scatter(ldst, [tail], iota + C)

                n = cnt[0]
                ng = (n + G - 1) // G

                def gbody(g, carry):
                    for j in range(G // _LN):
                        sidx[pl.ds(j * _LN, _LN)] = lsrc[pl.ds(g * G + j * _LN, _LN)]
                        didx[pl.ds(j * _LN, _LN)] = ldst[pl.ds(g * G + j * _LN, _LN)]
                    descs = [pltpu.async_copy(hs[b].at[sidx], stages[b], sem)
                             for b in range(NB)]
                    for d in descs:
                        d.wait()
                    for b in range(NB):
                        pltpu.sync_copy(stages[b], accs[b].at[didx], add=True)
                    return carry

                lax.fori_loop(0, ng, gbody, 0)

            plsc.subcore_barrier()
            for b in range(NB):
                pltpu.sync_copy(accs[b].at[pl.ds(r0, R)],
                                outs[b].at[pl.ds(lo + r0, R)])
            plsc.subcore_barrier()

    return functools.partial(
        pl.kernel,
        out_type=tuple(jax.ShapeDtypeStruct((_NP, _CB), jnp.float32)
                       for _ in range(NB)),
        mesh=_mesh(),
        compiler_params=pltpu.CompilerParams(needs_layout_passes=False),
        scratch_types=(
            [pltpu.VMEM((RND,), jnp.int32),
             pltpu.VMEM((RND,), jnp.int32),
             pltpu.VMEM((RND + G,), jnp.int32),
             pltpu.VMEM((RND + G,), jnp.int32),
             pltpu.VMEM((G,), jnp.int32),
             pltpu.VMEM((G,), jnp.int32)]
            + [pltpu.VMEM((G, _CB), jnp.float32) for _ in range(NB)]
            + [pltpu.VMEM_SHARED((C + 16, _CB), jnp.float32) for _ in range(NB)]
            + [pltpu.SemaphoreType.DMA]
        ),
    )(body)


_agg1_call = _make_agg(4, 1024, 5, 64)
_agg2_call = _make_agg(2, 1024, 5, 64)


_BR = 256


def _b1_kernel(x_ref, w_ref, d0_ref, d1_ref, h0, h1, h2, h3, dis_ref):
    deg = d0_ref[...] + d1_ref[...] + 1.0
    dis = lax.rsqrt(deg)
    dis_ref[...] = dis
    h = jnp.dot(x_ref[...] * dis, w_ref[...],
                preferred_element_type=jnp.float32)
    h0[...] = h[:, 0 * _CB:1 * _CB]
    h1[...] = h[:, 1 * _CB:2 * _CB]
    h2[...] = h[:, 2 * _CB:3 * _CB]
    h3[...] = h[:, 3 * _CB:4 * _CB]


def _b1_call(xp, W1, d0, d1):
    blk = pl.BlockSpec((_BR, _CB), lambda i: (i, 0))
    return pl.pallas_call(
        _b1_kernel,
        grid=(_NP // _BR,),
        in_specs=[
            pl.BlockSpec((_BR, _DF), lambda i: (i, 0)),
            pl.BlockSpec((_DF, _DH), lambda i: (0, 0)),
            pl.BlockSpec((_BR, 1), lambda i: (i, 0)),
            pl.BlockSpec((_BR, 1), lambda i: (i, 0)),
        ],
        out_specs=[blk, blk, blk, blk,
                   pl.BlockSpec((_BR, 1), lambda i: (i, 0))],
        out_shape=[jax.ShapeDtypeStruct((_NP, _CB), jnp.float32)] * 4
        + [jax.ShapeDtypeStruct((_NP, 1), jnp.float32)],
    )(xp, W1, d0, d1)


def _b2_kernel(a0, a1, a2, a3, dis_ref, b_ref, w_ref, o0, o1):
    dis = dis_ref[...]
    zs = []
    for b, a in enumerate((a0, a1, a2, a3)):
        t = dis * a[...] + b_ref[:, b * _CB:(b + 1) * _CB]
        zs.append(jnp.maximum(t, 0.0) * dis)
    z = jnp.concatenate(zs, axis=1)
    o = jnp.dot(z, w_ref[...], preferred_element_type=jnp.float32)
    o0[...] = o[:, 0 * _CB:1 * _CB]
    o1[...] = o[:, 1 * _CB:2 * _CB]


def _b2_call(agg1, dis, b1r, W2):
    blk = pl.BlockSpec((_BR, _CB), lambda i: (i, 0))
    return pl.pallas_call(
        _b2_kernel,
        grid=(_NP // _BR,),
        in_specs=[blk] * 4 + [
            pl.BlockSpec((_BR, 1), lambda i: (i, 0)),
            pl.BlockSpec((1, _DH), lambda i: (0, 0)),
            pl.BlockSpec((_DH, _DF), lambda i: (0, 0)),
        ],
        out_specs=[blk, blk],
        out_shape=[jax.ShapeDtypeStruct((_NP, _CB), jnp.float32)] * 2,
    )(*agg1, dis, b1r, W2)


def _b3_kernel(a0, a1, dis_ref, b_ref, o_ref):
    dis = dis_ref[...]
    o_ref[...] = jnp.concatenate(
        [dis * a0[...], dis * a1[...]], axis=1) + b_ref[...]


def _b3_call(agg2, dis, b2r):
    blk = pl.BlockSpec((_BR, _CB), lambda i: (i, 0))
    return pl.pallas_call(
        _b3_kernel,
        grid=(_NP // _BR,),
        in_specs=[blk] * 2 + [
            pl.BlockSpec((_BR, 1), lambda i: (i, 0)),
            pl.BlockSpec((1, _DF), lambda i: (0, 0)),
        ],
        out_specs=pl.BlockSpec((_BR, _DF), lambda i: (i, 0)),
        out_shape=jax.ShapeDtypeStruct((_NP, _DF), jnp.float32),
    )(*agg2, dis, b2r)


@jax.jit
def kernel(x, edge_index, W1, b1, W2, b2):
    src = edge_index[0].astype(jnp.int32)
    dst = edge_index[1].astype(jnp.int32)
    srcp = jnp.concatenate([src, jnp.zeros((_EP - _E,), jnp.int32)])
    dstp = jnp.concatenate([dst, jnp.full((_EP - _E,), _TRASH, jnp.int32)])
    xp = jnp.concatenate([x, jnp.zeros((_NP - _N, _DF), x.dtype)])

    degp = _deg_call(dstp)
    d0 = degp[:_NP, None]
    d1 = degp[_ND:_ND + _NP, None]

    *h1s, dis = _b1_call(xp, W1, d0, d1)
    agg1 = _agg1_call(srcp, dstp, *h1s)
    h2s = _b2_call(agg1, dis, b1[None, :], W2)
    agg2 = _agg2_call(srcp, dstp, *h2s)
    out = _b3_call(agg2, dis, b2[None, :])
    return out[:_N]


# speedup vs baseline: 3.4051x; 1.0265x over previous
import functools

import jax
import jax.numpy as jnp
from jax import lax
from jax.experimental import pallas as pl
from jax.experimental.pallas import tpu as pltpu
from jax.experimental.pallas import tpu_sc as plsc

_N = 10000
_E = 160000
_DF = 256
_DH = 512

_NC = 2
_NS = 16
_LN = 16
_CB = 128

_NP = 10240
_EP = 163840
_ND = 10496
_TRASH = _NP


def _mesh():
    return plsc.VectorSubcoreMesh(
        core_axis_name="c", subcore_axis_name="s",
        num_cores=_NC, num_subcores=_NS)


_DEG_EPT = _EP // (_NC * _NS)
_DEG_G = 128
_DEG_ZR = _ND // _NS


def _deg_body(dst_hbm, degp_hbm, dst_v, idx_v, ones_v, zbuf_v, deg_sh):
    c = lax.axis_index("c")
    s = lax.axis_index("s")
    wid = c * _NS + s
    zero = jnp.zeros((_LN,), jnp.float32)
    for j in range(_DEG_ZR // _LN):
        zbuf_v[pl.ds(j * _LN, _LN)] = zero
    one = jnp.ones((_LN,), jnp.float32)
    for j in range(_DEG_G // _LN):
        ones_v[pl.ds(j * _LN, _LN)] = one
    pltpu.sync_copy(zbuf_v, deg_sh.at[pl.ds(s * _DEG_ZR, _DEG_ZR)])
    plsc.subcore_barrier()

    for r in range(_DEG_EPT // 1024):
        pltpu.sync_copy(
            dst_hbm.at[pl.ds(wid * _DEG_EPT + r * 1024, 1024)], dst_v)

        def body(g, carry):
            for j in range(_DEG_G // _LN):
                idx_v[pl.ds(j * _LN, _LN)] = \
                    dst_v[pl.ds(g * _DEG_G + j * _LN, _LN)]
            pltpu.sync_copy(ones_v, deg_sh.at[idx_v], add=True)
            return carry

        lax.fori_loop(0, 1024 // _DEG_G, body, 0)
    plsc.subcore_barrier()
    pltpu.sync_copy(deg_sh.at[pl.ds(s * _DEG_ZR, _DEG_ZR)], zbuf_v)
    pltpu.sync_copy(zbuf_v, degp_hbm.at[pl.ds(c * _ND + s * _DEG_ZR, _DEG_ZR)])


def _make_deg():
    return functools.partial(
        pl.kernel,
        out_type=jax.ShapeDtypeStruct((_NC * _ND,), jnp.float32),
        mesh=_mesh(),
        compiler_params=pltpu.CompilerParams(needs_layout_passes=False),
        scratch_types=[
            pltpu.VMEM((1024,), jnp.int32),
            pltpu.VMEM((_DEG_G,), jnp.int32),
            pltpu.VMEM((_DEG_G,), jnp.float32),
            pltpu.VMEM((_DEG_ZR,), jnp.float32),
            pltpu.VMEM_SHARED((_ND,), jnp.float32),
        ],
    )(_deg_body)


_deg_call = _make_deg()


def _make_agg(NB, C, K, G=32, RND=2048):
    ET = _EP // _NS
    R = C // _NS
    RG = 16 if R % 16 == 0 else 8
    NRND = ET // RND

    def body(src_hbm, dst_hbm, *refs):
        hs = refs[:NB]
        outs = refs[NB:2 * NB]
        ebs, ebd, lsrc, ldst, sidx, didx = refs[2 * NB:2 * NB + 6]
        stages = refs[2 * NB + 6:2 * NB + 6 + NB]
        accs = refs[2 * NB + 6 + NB:2 * NB + 6 + 2 * NB]
        sem = refs[-1]

        c = lax.axis_index("c")
        s = lax.axis_index("s")
        base = c * (K * C)
        r0 = s * R
        iota = jnp.arange(_LN, dtype=jnp.int32)

        for k in range(K):
            lo = base + k * C
            for b in range(NB):
                pltpu.sync_copy(hs[b].at[pl.ds(lo + r0, R)],
                                accs[b].at[pl.ds(r0, R)])
            plsc.subcore_barrier()

            for r in range(NRND):
                e0 = s * ET + r * RND
                pltpu.sync_copy(src_hbm.at[pl.ds(e0, RND)], ebs)
                pltpu.sync_copy(dst_hbm.at[pl.ds(e0, RND)], ebd)

                def cbody(i, cnt):
                    sv = ebs[pl.ds(i * _LN, _LN)]
                    dv = ebd[pl.ds(i * _LN, _LN)]
                    m = (dv >= lo) & (dv < lo + C)
                    inc = plsc.cumsum(jnp.where(m, 1, 0).astype(jnp.int32))
                    pos = cnt + inc - 1
                    plsc.store_scatter(lsrc, [pos], sv, mask=m)
                    plsc.store_scatter(ldst, [pos], dv - lo, mask=m)
                    return cnt + plsc.all_reduce_population_count(m)

                cnt = lax.fori_loop(0, RND // _LN, cbody,
                                    jnp.zeros((_LN,), jnp.int32))
                for j in range(G // _LN):
                    tail = cnt + iota + j * _LN
                    plsc.store_scatter(lsrc, [tail], iota + j * _LN)
                    plsc.store_scatter(ldst, [tail], iota + C)

                n = cnt[0]
                ng = (n + G - 1) // G

                def gbody(g, carry):
                    for j in range(G // _LN):
                        sidx[pl.ds(j * _LN, _LN)] = lsrc[pl.ds(g * G + j * _LN, _LN)]
                        didx[pl.ds(j * _LN, _LN)] = ldst[pl.ds(g * G + j * _LN, _LN)]
                    descs = [pltpu.async_copy(hs[b].at[sidx], stages[b], sem)
                             for b in range(NB)]
                    for d in descs:
                        d.wait()
                    sdescs = [pltpu.async_copy(stages[b], accs[b].at[didx],
                                               sem, add=True)
                              for b in range(NB)]
                    for d in sdescs:
                        d.wait()
                    return carry

                lax.fori_loop(0, ng, gbody, 0)

            plsc.subcore_barrier()
            for b in range(NB):
                pltpu.sync_copy(accs[b].at[pl.ds(r0, R)],
                                outs[b].at[pl.ds(lo + r0, R)])
            plsc.subcore_barrier()

    return functools.partial(
        pl.kernel,
        out_type=tuple(jax.ShapeDtypeStruct((_NP, _CB), jnp.float32)
                       for _ in range(NB)),
        mesh=_mesh(),
        compiler_params=pltpu.CompilerParams(needs_layout_passes=False),
        scratch_types=(
            [pltpu.VMEM((RND,), jnp.int32),
             pltpu.VMEM((RND,), jnp.int32),
             pltpu.VMEM((RND + G,), jnp.int32),
             pltpu.VMEM((RND + G,), jnp.int32),
             pltpu.VMEM((G,), jnp.int32),
             pltpu.VMEM((G,), jnp.int32)]
            + [pltpu.VMEM((G, _CB), jnp.float32) for _ in range(NB)]
            + [pltpu.VMEM_SHARED((C + 16, _CB), jnp.float32) for _ in range(NB)]
            + [pltpu.SemaphoreType.DMA]
        ),
    )(body)


_agg1_call = _make_agg(4, 1024, 5, 64)
_agg2_call = _make_agg(2, 1024, 5, 64)


_BR = 256


def _b1_kernel(x_ref, w_ref, d0_ref, d1_ref, h0, h1, h2, h3, dis_ref):
    deg = d0_ref[...] + d1_ref[...] + 1.0
    dis = lax.rsqrt(deg)
    dis_ref[...] = dis
    h = jnp.dot(x_ref[...] * dis, w_ref[...],
                preferred_element_type=jnp.float32)
    h0[...] = h[:, 0 * _CB:1 * _CB]
    h1[...] = h[:, 1 * _CB:2 * _CB]
    h2[...] = h[:, 2 * _CB:3 * _CB]
    h3[...] = h[:, 3 * _CB:4 * _CB]


def _b1_call(xp, W1, d0, d1):
    blk = pl.BlockSpec((_BR, _CB), lambda i: (i, 0))
    return pl.pallas_call(
        _b1_kernel,
        grid=(_NP // _BR,),
        in_specs=[
            pl.BlockSpec((_BR, _DF), lambda i: (i, 0)),
            pl.BlockSpec((_DF, _DH), lambda i: (0, 0)),
            pl.BlockSpec((_BR, 1), lambda i: (i, 0)),
            pl.BlockSpec((_BR, 1), lambda i: (i, 0)),
        ],
        out_specs=[blk, blk, blk, blk,
                   pl.BlockSpec((_BR, 1), lambda i: (i, 0))],
        out_shape=[jax.ShapeDtypeStruct((_NP, _CB), jnp.float32)] * 4
        + [jax.ShapeDtypeStruct((_NP, 1), jnp.float32)],
    )(xp, W1, d0, d1)


def _b2_kernel(a0, a1, a2, a3, dis_ref, b_ref, w_ref, o0, o1):
    dis = dis_ref[...]
    zs = []
    for b, a in enumerate((a0, a1, a2, a3)):
        t = dis * a[...] + b_ref[:, b * _CB:(b + 1) * _CB]
        zs.append(jnp.maximum(t, 0.0) * dis)
    z = jnp.concatenate(zs, axis=1)
    o = jnp.dot(z, w_ref[...], preferred_element_type=jnp.float32)
    o0[...] = o[:, 0 * _CB:1 * _CB]
    o1[...] = o[:, 1 * _CB:2 * _CB]


def _b2_call(agg1, dis, b1r, W2):
    blk = pl.BlockSpec((_BR, _CB), lambda i: (i, 0))
    return pl.pallas_call(
        _b2_kernel,
        grid=(_NP // _BR,),
        in_specs=[blk] * 4 + [
            pl.BlockSpec((_BR, 1), lambda i: (i, 0)),
            pl.BlockSpec((1, _DH), lambda i: (0, 0)),
            pl.BlockSpec((_DH, _DF), lambda i: (0, 0)),
        ],
        out_specs=[blk, blk],
        out_shape=[jax.ShapeDtypeStruct((_NP, _CB), jnp.float32)] * 2,
    )(*agg1, dis, b1r, W2)


def _b3_kernel(a0, a1, dis_ref, b_ref, o_ref):
    dis = dis_ref[...]
    o_ref[...] = jnp.concatenate(
        [dis * a0[...], dis * a1[...]], axis=1) + b_ref[...]


def _b3_call(agg2, dis, b2r):
    blk = pl.BlockSpec((_BR, _CB), lambda i: (i, 0))
    return pl.pallas_call(
        _b3_kernel,
        grid=(_NP // _BR,),
        in_specs=[blk] * 2 + [
            pl.BlockSpec((_BR, 1), lambda i: (i, 0)),
            pl.BlockSpec((1, _DF), lambda i: (0, 0)),
        ],
        out_specs=pl.BlockSpec((_BR, _DF), lambda i: (i, 0)),
        out_shape=jax.ShapeDtypeStruct((_NP, _DF), jnp.float32),
    )(*agg2, dis, b2r)


@jax.jit
def kernel(x, edge_index, W1, b1, W2, b2):
    src = edge_index[0].astype(jnp.int32)
    dst = edge_index[1].astype(jnp.int32)
    srcp = jnp.concatenate([src, jnp.zeros((_EP - _E,), jnp.int32)])
    dstp = jnp.concatenate([dst, jnp.full((_EP - _E,), _TRASH, jnp.int32)])
    xp = jnp.concatenate([x, jnp.zeros((_NP - _N, _DF), x.dtype)])

    degp = _deg_call(dstp)
    d0 = degp[:_NP, None]
    d1 = degp[_ND:_ND + _NP, None]

    *h1s, dis = _b1_call(xp, W1, d0, d1)
    agg1 = _agg1_call(srcp, dstp, *h1s)
    h2s = _b2_call(agg1, dis, b1[None, :], W2)
    agg2 = _agg2_call(srcp, dstp, *h2s)
    out = _b3_call(agg2, dis, b2[None, :])
    return out[:_N]
